# Initial kernel scaffold; baseline (speedup 1.0000x reference)
import jax
import jax.numpy as jnp
from jax.experimental import pallas as pl


def kernel(x, static_features, edge_index, tgcn1, tgcn2, lin_W, lin_b):
    raise NotImplementedError("write your pallas kernel here")


# speedup vs baseline: 19.1560x; 19.1560x over previous
import functools

import jax
import jax.numpy as jnp
from jax import lax
from jax.experimental import pallas as pl
from jax.experimental.pallas import tpu as pltpu
from jax.experimental.pallas import tpu_sc as plsc

N = 10000
E = 320000
WIN = 10
EMB = 118
H1 = 128
H2 = 64
F = 128

NC = 2
NS = 16
NW = NC * NS
EPW = E // NW
CH = 80
NCHUNK = EPW // CH
NWRITE = 10
RPW = N // NWRITE
ZR = 200

NP = 10240
HR = NP // 128

_SC_CACHE = {}


def _sc_kernels():
    if "prop" in _SC_CACHE:
        return _SC_CACHE["prop"]

    mesh = plsc.VectorSubcoreMesh(core_axis_name="c", subcore_axis_name="s")

    @functools.partial(
        pl.kernel,
        out_type=jax.ShapeDtypeStruct((NC, N, F), jnp.float32),
        mesh=mesh,
        scratch_types=[
            pltpu.VMEM((CH,), jnp.int32),
            pltpu.VMEM((CH,), jnp.int32),
            pltpu.VMEM((CH, F), jnp.float32),
            pltpu.VMEM((ZR, F), jnp.float32),
            pltpu.VMEM_SHARED((N, F), jnp.float32),
            pltpu.SemaphoreType.DMA,
        ],
    )
    def _prop(y_hbm, src_hbm, dst_hbm, out_hbm,
              src_v, dst_v, rows_v, zero_v, acc_sh, sem):
        c = lax.axis_index("c")
        s = lax.axis_index("s")
        wid = s * NC + c

        def fill_zero(i, _):
            r = i // 8
            k = i % 8
            zero_v[r, pl.ds(k * 16, 16)] = jnp.zeros((16,), jnp.float32)
            return 0

        lax.fori_loop(0, ZR * 8, fill_zero, 0)

        @pl.when(s < NWRITE)
        def _():
            for j in range(RPW // ZR):
                pltpu.sync_copy(zero_v, acc_sh.at[pl.ds(s * RPW + j * ZR, ZR)])

        plsc.subcore_barrier()

        base = wid * EPW

        def body(i, _):
            off = base + i * CH
            pltpu.sync_copy(src_hbm.at[pl.ds(off, CH)], src_v)
            pltpu.async_copy(y_hbm.at[src_v], rows_v, sem).wait()
            pltpu.sync_copy(dst_hbm.at[pl.ds(off, CH)], dst_v)
            pltpu.sync_copy(rows_v, acc_sh.at[dst_v], add=True)
            return 0

        lax.fori_loop(0, NCHUNK, body, 0)
        plsc.subcore_barrier()

        @pl.when(s < NWRITE)
        def _():
            pltpu.sync_copy(acc_sh.at[pl.ds(s * RPW, RPW)],
                            out_hbm.at[c, pl.ds(s * RPW, RPW)])

    _SC_CACHE["prop"] = _prop
    return _prop


_B = 1000
_EPAD = 327680
_ER = 256
_EB = _ER * 128


def _deg_body(dst_ref, dis_ref, acc_ref):
    i = pl.program_id(0)

    @pl.when(i == 0)
    def _():
        acc_ref[...] = jnp.zeros((HR, 128), jnp.float32)

    d = dst_ref[...].reshape(_EB)
    lo = d & 127
    hi = d >> 7
    ohlo = (lo[:, None] == lax.broadcasted_iota(jnp.int32, (_EB, 128), 1))
    ohhi = (hi[:, None] == lax.broadcasted_iota(jnp.int32, (_EB, HR), 1))
    acc_ref[...] += lax.dot_general(
        ohhi.astype(jnp.bfloat16), ohlo.astype(jnp.bfloat16),
        (((0,), (0,)), ((), ())), preferred_element_type=jnp.float32)

    @pl.when(i == pl.num_programs(0) - 1)
    def _():
        dis_ref[...] = lax.rsqrt(acc_ref[...] + 1.0)


def _deg_call(dst):
    dst2d = jnp.concatenate(
        [dst, jnp.full((_EPAD - E,), N, jnp.int32)]).reshape(_EPAD // 128, 128)
    grid = (_EPAD // _EB,)
    return pl.pallas_call(
        _deg_body,
        grid=grid,
        in_specs=[pl.BlockSpec((_ER, 128), lambda i: (i, 0))],
        out_specs=pl.BlockSpec((HR, 128), lambda i: (0, 0)),
        out_shape=jax.ShapeDtypeStruct((HR, 128), jnp.float32),
        scratch_shapes=[pltpu.VMEM((HR, 128), jnp.float32)],
    )(dst2d)


def _scale_body(dis_ref, comb_ref, y_ref):
    y_ref[...] = comb_ref[...] * dis_ref[...]


def _scale_call(dis, comb):
    grid = (N // _B,)
    return pl.pallas_call(
        _scale_body,
        grid=grid,
        in_specs=[
            pl.BlockSpec((_B, 1), lambda i: (i, 0)),
            pl.BlockSpec((_B, F), lambda i: (i, 0)),
        ],
        out_specs=pl.BlockSpec((_B, F), lambda i: (i, 0)),
        out_shape=jax.ShapeDtypeStruct((N, F), jnp.float32),
    )(dis, comb)


def _dot(a, b):
    return jnp.dot(a, b, preferred_element_type=jnp.float32,
                   precision=lax.Precision.HIGHEST)


def _layer_body(acc_ref, y_ref, dis_ref, wzt, bz, azt, abz, wht, bh, aht, abh,
                h_ref, y2_ref):
    dis = dis_ref[...]
    p = (acc_ref[0] + acc_ref[1] + y_ref[...]) * dis
    cz = _dot(p, wzt[...]) + bz[...]
    zg = jax.nn.sigmoid(_dot(cz, azt[...]) + abz[...])
    chh = _dot(p, wht[...]) + bh[...]
    ht = jnp.tanh(_dot(chh, aht[...]) + abh[...])
    h = jax.nn.relu((1.0 - zg) * ht)
    h_ref[...] = h
    y2_ref[...] = h * dis


def _layer_call(acc, y, dis, wzt, bz, azt, abz, wht, bh, aht, abh, hdim):
    grid = (N // _B,)
    full = lambda shape: pl.BlockSpec(shape, lambda i: tuple(0 for _ in shape))
    return pl.pallas_call(
        _layer_body,
        grid=grid,
        in_specs=[
            pl.BlockSpec((NC, _B, F), lambda i: (0, i, 0)),
            pl.BlockSpec((_B, F), lambda i: (i, 0)),
            pl.BlockSpec((_B, 1), lambda i: (i, 0)),
            full((F, hdim)), full((hdim,)), full((hdim, hdim)), full((hdim,)),
            full((F, hdim)), full((hdim,)), full((hdim, hdim)), full((hdim,)),
        ],
        out_specs=[
            pl.BlockSpec((_B, hdim), lambda i: (i, 0)),
            pl.BlockSpec((_B, hdim), lambda i: (i, 0)),
        ],
        out_shape=[
            jax.ShapeDtypeStruct((N, hdim), jnp.float32),
            jax.ShapeDtypeStruct((N, hdim), jnp.float32),
        ],
    )(acc, y, dis, wzt, bz, azt, abz, wht, bh, aht, abh)


def _final_body(h2_ref, lw_ref, lb_ref, out_ref):
    out_ref[...] = (jnp.sum(h2_ref[...] * lw_ref[0][None, :], axis=1,
                            keepdims=True) + lb_ref[0])


def _final_call(h2, lin_W, lin_b):
    grid = (N // _B,)
    return pl.pallas_call(
        _final_body,
        grid=grid,
        in_specs=[
            pl.BlockSpec((_B, H2), lambda i: (i, 0)),
            pl.BlockSpec((1, H2), lambda i: (0, 0)),
            pl.BlockSpec((1,), lambda i: (0,)),
        ],
        out_specs=pl.BlockSpec((_B, 1), lambda i: (i, 0)),
        out_shape=jax.ShapeDtypeStruct((N, 1), jnp.float32),
    )(h2, lin_W, lin_b)


def _layer_weights(p, hdim):
    return (p["conv_z_W"].T, p["conv_z_b"],
            p["lin_z_W"][:, :hdim].T, p["lin_z_b"],
            p["conv_h_W"].T, p["conv_h_b"],
            p["lin_h_W"][:, :hdim].T, p["lin_h_b"])


def kernel(x, static_features, edge_index, tgcn1, tgcn2, lin_W, lin_b):
    src = edge_index[0]
    dst = edge_index[1]
    comb = jnp.concatenate([x, static_features], axis=1)

    dis2d = _deg_call(dst)
    dis = dis2d.reshape(NP)[:N].reshape(N, 1)

    _prop = _sc_kernels()
    y1 = _scale_call(dis, comb)
    acc1 = _prop(y1, src, dst)
    h1, y2 = _layer_call(acc1, y1, dis, *_layer_weights(tgcn1, H1), hdim=H1)
    acc2 = _prop(y2, src, dst)
    h2, _y3 = _layer_call(acc2, y2, dis, *_layer_weights(tgcn2, H2), hdim=H2)
    out = _final_call(h2, lin_W, lin_b)[:, 0]
    return (out, h1, h2)


# speedup vs baseline: 33.3861x; 1.7429x over previous
import functools

import jax
import jax.numpy as jnp
from jax import lax
from jax.experimental import pallas as pl
from jax.experimental.pallas import tpu as pltpu
from jax.experimental.pallas import tpu_sc as plsc

N = 10000
E = 320000
WIN = 10
EMB = 118
H1 = 128
H2 = 64
F = 128

NC = 2
NS = 16
NW = NC * NS
EPW = E // NW
CH = 125
NCHUNK = EPW // CH
NPASS = 4
PCH = NCHUNK // NPASS
NBUF = 2
NOUT = PCH // NBUF
NWRITE = 10
RPW = N // NWRITE
ZR = 40

NP = 10240
HR = NP // 128

_SC_CACHE = {}


def _sc_kernels():
    if "prop" in _SC_CACHE:
        return _SC_CACHE["prop"]

    mesh = plsc.VectorSubcoreMesh(core_axis_name="c", subcore_axis_name="s")

    @functools.partial(
        pl.kernel,
        out_type=jax.ShapeDtypeStruct((NC, N, F), jnp.float32),
        mesh=mesh,
        scratch_types=[
            pltpu.VMEM((PCH, CH), jnp.int32),
            pltpu.VMEM((PCH, CH), jnp.int32),
            pltpu.VMEM((NBUF, CH, F), jnp.float32),
            pltpu.VMEM((ZR, F), jnp.float32),
            pltpu.VMEM_SHARED((N, F), jnp.float32),
            pltpu.SemaphoreType.DMA,
            pltpu.SemaphoreType.DMA,
        ],
    )
    def _prop(y_hbm, src_hbm, dst_hbm, out_hbm,
              src_v, dst_v, rows_v, zero_v, acc_sh, g0, g1):
        c = lax.axis_index("c")
        s = lax.axis_index("s")
        wid = s * NC + c
        gsem = (g0, g1)

        def fill_zero(i, _):
            r = i // 8
            k = i % 8
            zero_v[r, pl.ds(k * 16, 16)] = jnp.zeros((16,), jnp.float32)
            return 0

        lax.fori_loop(0, ZR * 8, fill_zero, 0)

        @pl.when(s < NWRITE)
        def _():
            for j in range(RPW // ZR):
                pltpu.sync_copy(zero_v, acc_sh.at[pl.ds(s * RPW + j * ZR, ZR)])

        plsc.subcore_barrier()

        for h in range(NPASS):
            pltpu.sync_copy(src_hbm.at[h, wid], src_v)
            pltpu.sync_copy(dst_hbm.at[h, wid], dst_v)

            for b in range(NBUF):
                pltpu.async_copy(y_hbm.at[src_v.at[b]], rows_v.at[b], gsem[b])

            def group(g, _):
                for b in range(NBUF):
                    i = g * NBUF + b
                    pltpu.make_async_copy(
                        y_hbm.at[src_v.at[i]], rows_v.at[b], gsem[b]).wait()
                    pltpu.sync_copy(rows_v.at[b], acc_sh.at[dst_v.at[i]],
                                    add=True)

                    @pl.when(g < NOUT - 1)
                    def _():
                        pltpu.async_copy(
                            y_hbm.at[src_v.at[i + NBUF]], rows_v.at[b],
                            gsem[b])
                return 0

            lax.fori_loop(0, NOUT, group, 0)

        plsc.subcore_barrier()

        @pl.when(s < NWRITE)
        def _():
            pltpu.sync_copy(acc_sh.at[pl.ds(s * RPW, RPW)],
                            out_hbm.at[c, pl.ds(s * RPW, RPW)])

    _SC_CACHE["prop"] = _prop
    return _prop


_B = 1000
_EPAD = 327680
_ER = 256
_EB = _ER * 128


def _deg_body(dst_ref, dis_ref, acc_ref):
    i = pl.program_id(0)

    @pl.when(i == 0)
    def _():
        acc_ref[...] = jnp.zeros((HR, 128), jnp.float32)

    d = dst_ref[...].reshape(_EB)
    lo = d & 127
    hi = d >> 7
    ohlo = (lo[:, None] == lax.broadcasted_iota(jnp.int32, (_EB, 128), 1))
    ohhi = (hi[:, None] == lax.broadcasted_iota(jnp.int32, (_EB, HR), 1))
    acc_ref[...] += lax.dot_general(
        ohhi.astype(jnp.bfloat16), ohlo.astype(jnp.bfloat16),
        (((0,), (0,)), ((), ())), preferred_element_type=jnp.float32)

    @pl.when(i == pl.num_programs(0) - 1)
    def _():
        dis_ref[...] = lax.rsqrt(acc_ref[...] + 1.0)


def _deg_call(dst):
    dst2d = jnp.concatenate(
        [dst, jnp.full((_EPAD - E,), N, jnp.int32)]).reshape(_EPAD // 128, 128)
    grid = (_EPAD // _EB,)
    return pl.pallas_call(
        _deg_body,
        grid=grid,
        in_specs=[pl.BlockSpec((_ER, 128), lambda i: (i, 0))],
        out_specs=pl.BlockSpec((HR, 128), lambda i: (0, 0)),
        out_shape=jax.ShapeDtypeStruct((HR, 128), jnp.float32),
        scratch_shapes=[pltpu.VMEM((HR, 128), jnp.float32)],
    )(dst2d)


def _scale_body(dis_ref, comb_ref, y_ref):
    y_ref[...] = comb_ref[...] * dis_ref[...]


def _scale_call(dis, comb):
    grid = (N // _B,)
    return pl.pallas_call(
        _scale_body,
        grid=grid,
        in_specs=[
            pl.BlockSpec((_B, 1), lambda i: (i, 0)),
            pl.BlockSpec((_B, F), lambda i: (i, 0)),
        ],
        out_specs=pl.BlockSpec((_B, F), lambda i: (i, 0)),
        out_shape=jax.ShapeDtypeStruct((N, F), jnp.float32),
    )(dis, comb)


def _dot(a, b):
    return jnp.dot(a, b, preferred_element_type=jnp.float32,
                   precision=lax.Precision.HIGHEST)


def _layer_body(acc_ref, y_ref, dis_ref, wzt, bz, azt, abz, wht, bh, aht, abh,
                h_ref, y2_ref):
    dis = dis_ref[...]
    p = (acc_ref[0] + acc_ref[1] + y_ref[...]) * dis
    cz = _dot(p, wzt[...]) + bz[...]
    zg = jax.nn.sigmoid(_dot(cz, azt[...]) + abz[...])
    chh = _dot(p, wht[...]) + bh[...]
    ht = jnp.tanh(_dot(chh, aht[...]) + abh[...])
    h = jax.nn.relu((1.0 - zg) * ht)
    h_ref[...] = h
    y2_ref[...] = h * dis


def _layer_call(acc, y, dis, wzt, bz, azt, abz, wht, bh, aht, abh, hdim):
    grid = (N // _B,)
    full = lambda shape: pl.BlockSpec(shape, lambda i: tuple(0 for _ in shape))
    return pl.pallas_call(
        _layer_body,
        grid=grid,
        in_specs=[
            pl.BlockSpec((NC, _B, F), lambda i: (0, i, 0)),
            pl.BlockSpec((_B, F), lambda i: (i, 0)),
            pl.BlockSpec((_B, 1), lambda i: (i, 0)),
            full((F, hdim)), full((hdim,)), full((hdim, hdim)), full((hdim,)),
            full((F, hdim)), full((hdim,)), full((hdim, hdim)), full((hdim,)),
        ],
        out_specs=[
            pl.BlockSpec((_B, hdim), lambda i: (i, 0)),
            pl.BlockSpec((_B, hdim), lambda i: (i, 0)),
        ],
        out_shape=[
            jax.ShapeDtypeStruct((N, hdim), jnp.float32),
            jax.ShapeDtypeStruct((N, hdim), jnp.float32),
        ],
    )(acc, y, dis, wzt, bz, azt, abz, wht, bh, aht, abh)


def _layer2_body(acc_ref, y_ref, dis_ref, wzt, bz, azt, abz, wht, bh, aht,
                 abh, lw_ref, lb_ref, h_ref, out_ref):
    dis = dis_ref[...]
    p = (acc_ref[0] + acc_ref[1] + y_ref[...]) * dis
    cz = _dot(p, wzt[...]) + bz[...]
    zg = jax.nn.sigmoid(_dot(cz, azt[...]) + abz[...])
    chh = _dot(p, wht[...]) + bh[...]
    ht = jnp.tanh(_dot(chh, aht[...]) + abh[...])
    h = jax.nn.relu((1.0 - zg) * ht)
    h_ref[...] = h
    out_ref[...] = (jnp.sum(h * lw_ref[0][None, :], axis=1,
                            keepdims=True) + lb_ref[0])


def _layer2_call(acc, y, dis, wzt, bz, azt, abz, wht, bh, aht, abh,
                 lin_W, lin_b):
    grid = (N // _B,)
    hdim = H2
    full = lambda shape: pl.BlockSpec(shape, lambda i: tuple(0 for _ in shape))
    return pl.pallas_call(
        _layer2_body,
        grid=grid,
        in_specs=[
            pl.BlockSpec((NC, _B, F), lambda i: (0, i, 0)),
            pl.BlockSpec((_B, F), lambda i: (i, 0)),
            pl.BlockSpec((_B, 1), lambda i: (i, 0)),
            full((F, hdim)), full((hdim,)), full((hdim, hdim)), full((hdim,)),
            full((F, hdim)), full((hdim,)), full((hdim, hdim)), full((hdim,)),
            full((1, hdim)), full((1,)),
        ],
        out_specs=[
            pl.BlockSpec((_B, hdim), lambda i: (i, 0)),
            pl.BlockSpec((_B, 1), lambda i: (i, 0)),
        ],
        out_shape=[
            jax.ShapeDtypeStruct((N, hdim), jnp.float32),
            jax.ShapeDtypeStruct((N, 1), jnp.float32),
        ],
    )(acc, y, dis, wzt, bz, azt, abz, wht, bh, aht, abh, lin_W, lin_b)


def _layer_weights(p, hdim):
    return (p["conv_z_W"].T, p["conv_z_b"],
            p["lin_z_W"][:, :hdim].T, p["lin_z_b"],
            p["conv_h_W"].T, p["conv_h_b"],
            p["lin_h_W"][:, :hdim].T, p["lin_h_b"])


def kernel(x, static_features, edge_index, tgcn1, tgcn2, lin_W, lin_b):
    src = edge_index[0]
    dst = edge_index[1]
    src3 = src.reshape(NW, NPASS, PCH, CH).transpose(1, 0, 2, 3)
    dst3 = dst.reshape(NW, NPASS, PCH, CH).transpose(1, 0, 2, 3)
    comb = jnp.concatenate([x, static_features], axis=1)

    dis2d = _deg_call(dst)
    dis = dis2d.reshape(NP)[:N].reshape(N, 1)

    _prop = _sc_kernels()
    y1 = _scale_call(dis, comb)
    acc1 = _prop(y1, src3, dst3)
    h1, y2 = _layer_call(acc1, y1, dis, *_layer_weights(tgcn1, H1), hdim=H1)
    acc2 = _prop(y2, src3, dst3)
    h2, out2d = _layer2_call(acc2, y2, dis, *_layer_weights(tgcn2, H2),
                             lin_W=lin_W, lin_b=lin_b)
    return (out2d[:, 0], h1, h2)


# speedup vs baseline: 47.4994x; 1.4227x over previous
import functools

import jax
import jax.numpy as jnp
from jax import lax
from jax.experimental import pallas as pl
from jax.experimental.pallas import tpu as pltpu
from jax.experimental.pallas import tpu_sc as plsc

N = 10000
E = 320000
WIN = 10
EMB = 118
H1 = 128
H2 = 64
F = 128

NC = 2
NS = 16
NW = NC * NS
EPW = E // NW
CH = 50
NCHUNK = EPW // CH
NPASS = 5
PCH = NCHUNK // NPASS
NBUF = 4
NOUT = PCH // NBUF
DCH = 125
DNCHUNK = EPW // DCH
DNPASS = 4
DPCH = DNCHUNK // DNPASS
NWRITE = 10
RPW = N // NWRITE
ZR = 40

NP = 10240
HR = NP // 128

_SC_CACHE = {}


def _sc_kernels():
    if "prop" in _SC_CACHE:
        return _SC_CACHE["deg"], _SC_CACHE["prop"]

    mesh = plsc.VectorSubcoreMesh(core_axis_name="c", subcore_axis_name="s")


    @functools.partial(
        pl.kernel,
        out_type=jax.ShapeDtypeStruct((NC, N, F), jnp.float32),
        mesh=mesh,
        scratch_types=[
            pltpu.VMEM((DNPASS, DPCH, DCH), jnp.int32),
            pltpu.VMEM((DCH, F), jnp.float32),
            pltpu.VMEM((ZR, F), jnp.float32),
            pltpu.VMEM_SHARED((N, F), jnp.float32),
            pltpu.SemaphoreType.DMA,
        ],
    )
    def _deg(dst_hbm, out_hbm, dst_v, ones_v, zero_v, acc_sh, sem):
        c = lax.axis_index("c")
        s = lax.axis_index("s")
        wid = s * NC + c

        def fill(i, _):
            r = i // 8
            k = i % 8
            zero_v[r, pl.ds(k * 16, 16)] = jnp.zeros((16,), jnp.float32)
            ones_v[r, pl.ds(k * 16, 16)] = jnp.ones((16,), jnp.float32)
            return 0

        lax.fori_loop(0, ZR * 8, fill, 0)

        def fill1(i, _):
            r = ZR + i // 8
            k = i % 8
            ones_v[r, pl.ds(k * 16, 16)] = jnp.ones((16,), jnp.float32)
            return 0

        lax.fori_loop(0, (DCH - ZR) * 8, fill1, 0)

        @pl.when(s < NWRITE)
        def _():
            for j in range(RPW // ZR):
                pltpu.sync_copy(zero_v, acc_sh.at[pl.ds(s * RPW + j * ZR, ZR)])

        plsc.subcore_barrier()

        for h in range(DNPASS):
            pltpu.sync_copy(dst_hbm.at[h, wid], dst_v.at[h])

        def fire(i, _):
            h = i // DPCH
            j = i - h * DPCH
            pltpu.async_copy(ones_v, acc_sh.at[dst_v.at[h, j]], sem, add=True)
            return 0

        lax.fori_loop(0, DNCHUNK, fire, 0)

        def drain(i, _):
            pltpu.make_async_copy(ones_v, acc_sh.at[dst_v.at[0, 0]],
                                  sem).wait()
            return 0

        lax.fori_loop(0, DNCHUNK, drain, 0)
        plsc.subcore_barrier()

        @pl.when(s < NWRITE)
        def _():
            pltpu.sync_copy(acc_sh.at[pl.ds(s * RPW, RPW)],
                            out_hbm.at[c, pl.ds(s * RPW, RPW)])


    @functools.partial(
        pl.kernel,
        out_type=jax.ShapeDtypeStruct((NC, N, F), jnp.float32),
        mesh=mesh,
        scratch_types=[
            pltpu.VMEM((PCH, CH), jnp.int32),
            pltpu.VMEM((PCH, CH), jnp.int32),
            pltpu.VMEM((NBUF, CH, F), jnp.float32),
            pltpu.VMEM((ZR, F), jnp.float32),
            pltpu.VMEM_SHARED((N, F), jnp.float32),
            pltpu.SemaphoreType.DMA,
            pltpu.SemaphoreType.DMA,
            pltpu.SemaphoreType.DMA,
            pltpu.SemaphoreType.DMA,
        ],
    )
    def _prop(y_hbm, src_hbm, dst_hbm, out_hbm,
              src_v, dst_v, rows_v, zero_v, acc_sh, g0, g1, g2, g3):
        c = lax.axis_index("c")
        s = lax.axis_index("s")
        wid = s * NC + c
        gsem = (g0, g1, g2, g3)

        def fill_zero(i, _):
            r = i // 8
            k = i % 8
            zero_v[r, pl.ds(k * 16, 16)] = jnp.zeros((16,), jnp.float32)
            return 0

        lax.fori_loop(0, ZR * 8, fill_zero, 0)

        @pl.when(s < NWRITE)
        def _():
            for j in range(RPW // ZR):
                pltpu.sync_copy(zero_v, acc_sh.at[pl.ds(s * RPW + j * ZR, ZR)])

        plsc.subcore_barrier()

        for h in range(NPASS):
            pltpu.sync_copy(src_hbm.at[h, wid], src_v)
            pltpu.sync_copy(dst_hbm.at[h, wid], dst_v)

            for b in range(NBUF):
                pltpu.async_copy(y_hbm.at[src_v.at[b]], rows_v.at[b], gsem[b])

            def group(g, _):
                for b in range(NBUF):
                    i = g * NBUF + b
                    pltpu.make_async_copy(
                        y_hbm.at[src_v.at[i]], rows_v.at[b], gsem[b]).wait()
                    pltpu.sync_copy(rows_v.at[b], acc_sh.at[dst_v.at[i]],
                                    add=True)

                    @pl.when(g < NOUT - 1)
                    def _():
                        pltpu.async_copy(
                            y_hbm.at[src_v.at[i + NBUF]], rows_v.at[b],
                            gsem[b])
                return 0

            lax.fori_loop(0, NOUT, group, 0)

        plsc.subcore_barrier()

        @pl.when(s < NWRITE)
        def _():
            pltpu.sync_copy(acc_sh.at[pl.ds(s * RPW, RPW)],
                            out_hbm.at[c, pl.ds(s * RPW, RPW)])

    _SC_CACHE["deg"] = _deg
    _SC_CACHE["prop"] = _prop
    return _deg, _prop


_B = 1000


def _scale_body(degp_ref, comb_ref, y_ref, dis_ref):
    deg = degp_ref[0, :, 0:1] + degp_ref[1, :, 0:1] + 1.0
    dis = lax.rsqrt(deg)
    dis_ref[...] = dis
    y_ref[...] = comb_ref[...] * dis


def _scale_call(degp, comb):
    grid = (N // _B,)
    return pl.pallas_call(
        _scale_body,
        grid=grid,
        in_specs=[
            pl.BlockSpec((NC, _B, F), lambda i: (0, i, 0)),
            pl.BlockSpec((_B, F), lambda i: (i, 0)),
        ],
        out_specs=[
            pl.BlockSpec((_B, F), lambda i: (i, 0)),
            pl.BlockSpec((_B, 1), lambda i: (i, 0)),
        ],
        out_shape=[
            jax.ShapeDtypeStruct((N, F), jnp.float32),
            jax.ShapeDtypeStruct((N, 1), jnp.float32),
        ],
    )(degp, comb)


def _dot(a, b):
    return jnp.dot(a, b, preferred_element_type=jnp.float32,
                   precision=lax.Precision.DEFAULT)


def _layer_body(acc_ref, y_ref, dis_ref, wzt, bz, azt, abz, wht, bh, aht, abh,
                h_ref, y2_ref):
    dis = dis_ref[...]
    p = (acc_ref[0] + acc_ref[1] + y_ref[...]) * dis
    cz = _dot(p, wzt[...]) + bz[...]
    zg = jax.nn.sigmoid(_dot(cz, azt[...]) + abz[...])
    chh = _dot(p, wht[...]) + bh[...]
    ht = jnp.tanh(_dot(chh, aht[...]) + abh[...])
    h = jax.nn.relu((1.0 - zg) * ht)
    h_ref[...] = h
    y2_ref[...] = h * dis


def _layer_call(acc, y, dis, wzt, bz, azt, abz, wht, bh, aht, abh, hdim):
    grid = (N // _B,)
    full = lambda shape: pl.BlockSpec(shape, lambda i: tuple(0 for _ in shape))
    return pl.pallas_call(
        _layer_body,
        grid=grid,
        in_specs=[
            pl.BlockSpec((NC, _B, F), lambda i: (0, i, 0)),
            pl.BlockSpec((_B, F), lambda i: (i, 0)),
            pl.BlockSpec((_B, 1), lambda i: (i, 0)),
            full((F, hdim)), full((hdim,)), full((hdim, hdim)), full((hdim,)),
            full((F, hdim)), full((hdim,)), full((hdim, hdim)), full((hdim,)),
        ],
        out_specs=[
            pl.BlockSpec((_B, hdim), lambda i: (i, 0)),
            pl.BlockSpec((_B, hdim), lambda i: (i, 0)),
        ],
        out_shape=[
            jax.ShapeDtypeStruct((N, hdim), jnp.float32),
            jax.ShapeDtypeStruct((N, hdim), jnp.float32),
        ],
    )(acc, y, dis, wzt, bz, azt, abz, wht, bh, aht, abh)


def _layer2_body(acc_ref, y_ref, dis_ref, wzt, bz, azt, abz, wht, bh, aht,
                 abh, lw_ref, lb_ref, h_ref, out_ref):
    dis = dis_ref[...]
    p = (acc_ref[0] + acc_ref[1] + y_ref[...]) * dis
    cz = _dot(p, wzt[...]) + bz[...]
    zg = jax.nn.sigmoid(_dot(cz, azt[...]) + abz[...])
    chh = _dot(p, wht[...]) + bh[...]
    ht = jnp.tanh(_dot(chh, aht[...]) + abh[...])
    h = jax.nn.relu((1.0 - zg) * ht)
    h_ref[...] = h
    out_ref[...] = _dot(h, lw_ref[...]) + lb_ref[0]


def _layer2_call(acc, y, dis, wzt, bz, azt, abz, wht, bh, aht, abh,
                 lin_W, lin_b):
    grid = (N // _B,)
    hdim = H2
    full = lambda shape: pl.BlockSpec(shape, lambda i: tuple(0 for _ in shape))
    return pl.pallas_call(
        _layer2_body,
        grid=grid,
        in_specs=[
            pl.BlockSpec((NC, _B, F), lambda i: (0, i, 0)),
            pl.BlockSpec((_B, F), lambda i: (i, 0)),
            pl.BlockSpec((_B, 1), lambda i: (i, 0)),
            full((F, hdim)), full((hdim,)), full((hdim, hdim)), full((hdim,)),
            full((F, hdim)), full((hdim,)), full((hdim, hdim)), full((hdim,)),
            full((hdim, 1)), full((1,)),
        ],
        out_specs=[
            pl.BlockSpec((_B, hdim), lambda i: (i, 0)),
            pl.BlockSpec((_B, 1), lambda i: (i, 0)),
        ],
        out_shape=[
            jax.ShapeDtypeStruct((N, hdim), jnp.float32),
            jax.ShapeDtypeStruct((N, 1), jnp.float32),
        ],
    )(acc, y, dis, wzt, bz, azt, abz, wht, bh, aht, abh, lin_W, lin_b)


def _layer_weights(p, hdim):
    return (p["conv_z_W"].T, p["conv_z_b"],
            p["lin_z_W"][:, :hdim].T, p["lin_z_b"],
            p["conv_h_W"].T, p["conv_h_b"],
            p["lin_h_W"][:, :hdim].T, p["lin_h_b"])


def kernel(x, static_features, edge_index, tgcn1, tgcn2, lin_W, lin_b):
    src = edge_index[0]
    dst = edge_index[1]
    src3 = src.reshape(NPASS, NW, PCH, CH)
    dst3 = dst.reshape(NPASS, NW, PCH, CH)
    dst3d = dst.reshape(DNPASS, NW, DPCH, DCH)
    comb = jnp.concatenate([x, static_features], axis=1)

    _deg, _prop = _sc_kernels()
    degp = _deg(dst3d)
    y1, dis = _scale_call(degp, comb)
    acc1 = _prop(y1, src3, dst3)
    h1, y2 = _layer_call(acc1, y1, dis, *_layer_weights(tgcn1, H1), hdim=H1)
    acc2 = _prop(y2, src3, dst3)
    h2, out2d = _layer2_call(acc2, y2, dis, *_layer_weights(tgcn2, H2),
                             lin_W=lin_W.T, lin_b=lin_b)
    return (out2d[:, 0], h1, h2)
